# baseline (device time: 31163 ns/iter reference)
import jax
import jax.numpy as jnp
from jax import lax
from jax.experimental import pallas as pl
from jax.experimental.pallas import tpu as pltpu

N_DEV = 4
B, SQ, SKV = 2, 256, 256
H_LOC, DH = 4, 64
D_MODEL = 512
D_CTX = H_LOC * DH
BLK = 64


def kernel(x, Wq, K_ext, V_ext, Wo):
    my = lax.axis_index("i")
    Wq_loc = lax.dynamic_slice_in_dim(Wq, my * D_CTX, D_CTX, axis=1)
    x2d = x.reshape(B * SQ, D_MODEL)
    Kt = K_ext.transpose(0, 2, 1, 3)
    Vt = V_ext.transpose(0, 2, 1, 3)

    def body(x_ref, wq_ref, k_ref, v_ref, wo_ref, out_ref,
             comm_ref, send_sems, recv_sems):
        my_pos = lax.axis_index("i")
        left = lax.rem(my_pos - 1 + N_DEV, N_DEV)
        right = lax.rem(my_pos + 1, N_DEV)

        barrier_sem = pltpu.get_barrier_semaphore()
        for nbr in (left, right):
            pl.semaphore_signal(
                barrier_sem, inc=1,
                device_id=(nbr,), device_id_type=pl.DeviceIdType.MESH,
            )
        pl.semaphore_wait(barrier_sem, 2)

        q2d = jnp.dot(x_ref[...], wq_ref[...],
                      preferred_element_type=jnp.float32)

        qb = lax.broadcasted_iota(jnp.int32, (SQ, SKV), 0) // BLK
        kb = lax.broadcasted_iota(jnp.int32, (SQ, SKV), 1) // BLK
        mask = kb <= qb

        for b in range(B):
            for h in range(H_LOC):
                q = q2d[b * SQ:(b + 1) * SQ, h * DH:(h + 1) * DH]
                s = lax.dot_general(
                    q, k_ref[b, h], (((1,), (1,)), ((), ())),
                    preferred_element_type=jnp.float32) * 0.125
                s = jnp.where(mask, s, -1e9)
                m = jnp.max(s, axis=1, keepdims=True)
                w = jnp.exp(s - m)
                w = w / jnp.sum(w, axis=1, keepdims=True)
                ctx_bh = jnp.dot(w, v_ref[b, h],
                                 preferred_element_type=jnp.float32)
                comm_ref[0, b * SQ:(b + 1) * SQ, h * DH:(h + 1) * DH] = ctx_bh

        acc = jnp.dot(comm_ref[0],
                      wo_ref[pl.ds(my_pos * D_CTX, D_CTX), :],
                      preferred_element_type=jnp.float32)

        for hop in range(N_DEV - 1):
            rdma = pltpu.make_async_remote_copy(
                src_ref=comm_ref.at[hop],
                dst_ref=comm_ref.at[hop + 1],
                send_sem=send_sems.at[hop],
                recv_sem=recv_sems.at[hop],
                device_id=(right,),
                device_id_type=pl.DeviceIdType.MESH,
            )
            rdma.start()
            rdma.wait()
            origin = lax.rem(my_pos - hop - 1 + N_DEV, N_DEV)
            acc = acc + jnp.dot(
                comm_ref[hop + 1],
                wo_ref[pl.ds(origin * D_CTX, D_CTX), :],
                preferred_element_type=jnp.float32)

        out_ref[...] = acc

    out2d = pl.pallas_call(
        body,
        out_shape=jax.ShapeDtypeStruct((B * SQ, D_MODEL), jnp.float32),
        in_specs=[pl.BlockSpec(memory_space=pltpu.VMEM)] * 5,
        out_specs=pl.BlockSpec(memory_space=pltpu.VMEM),
        scratch_shapes=[
            pltpu.VMEM((N_DEV, B * SQ, D_CTX), jnp.float32),
            pltpu.SemaphoreType.DMA((N_DEV - 1,)),
            pltpu.SemaphoreType.DMA((N_DEV - 1,)),
        ],
        compiler_params=pltpu.CompilerParams(collective_id=0),
    )(x2d, Wq_loc, Kt, Vt, Wo)
    return out2d.reshape(B, SQ, D_MODEL)


# device time: 15809 ns/iter; 1.9712x vs baseline; 1.9712x over previous
import jax
import jax.numpy as jnp
from jax import lax
from jax.experimental import pallas as pl
from jax.experimental.pallas import tpu as pltpu

N_DEV = 4
B, SQ, SKV = 2, 256, 256
H_LOC, DH = 4, 64
D_MODEL = 512
D_CTX = H_LOC * DH
HALF = D_CTX // 2
BLK = 64


def kernel(x, Wq, K_ext, V_ext, Wo):
    my = lax.axis_index("i")
    Wq_loc = lax.dynamic_slice_in_dim(Wq, my * D_CTX, D_CTX, axis=1)
    x2d = x.reshape(B * SQ, D_MODEL)
    Kt = K_ext.transpose(0, 2, 1, 3)
    Vt = V_ext.transpose(0, 2, 1, 3)

    def body(x_ref, wq_ref, k_ref, v_ref, wo_ref, out_ref,
             own_ref, chunk_l_ref, chunk_r_ref, half_cw_ref, half_ccw_ref,
             send_sems, recv_sems):
        my_pos = lax.axis_index("i")
        left = lax.rem(my_pos - 1 + N_DEV, N_DEV)
        right = lax.rem(my_pos + 1, N_DEV)

        barrier_sem = pltpu.get_barrier_semaphore()
        for nbr in (left, right):
            pl.semaphore_signal(
                barrier_sem, inc=1,
                device_id=(nbr,), device_id_type=pl.DeviceIdType.MESH,
            )

        q2d = jnp.dot(x_ref[...], wq_ref[...],
                      preferred_element_type=jnp.float32)

        qb = lax.broadcasted_iota(jnp.int32, (SQ, SKV), 0) // BLK
        kb = lax.broadcasted_iota(jnp.int32, (SQ, SKV), 1) // BLK
        mask = kb <= qb

        ctx_rows = []
        for b in range(B):
            per_h = []
            for h in range(H_LOC):
                q = q2d[b * SQ:(b + 1) * SQ, h * DH:(h + 1) * DH]
                s = lax.dot_general(
                    q, k_ref[b, h], (((1,), (1,)), ((), ())),
                    preferred_element_type=jnp.float32) * 0.125
                s = jnp.where(mask, s, -1e9)
                m = jnp.max(s, axis=1, keepdims=True)
                w = jnp.exp(s - m)
                w = w / jnp.sum(w, axis=1, keepdims=True)
                per_h.append(jnp.dot(w, v_ref[b, h],
                                     preferred_element_type=jnp.float32))
            ctx_rows.append(jnp.concatenate(per_h, axis=1))
        ctx = jnp.concatenate(ctx_rows, axis=0)

        ctx_bf = ctx.astype(jnp.bfloat16)
        own_ref[0] = ctx_bf[:, :HALF]
        own_ref[1] = ctx_bf[:, HALF:]

        pl.semaphore_wait(barrier_sem, 2)

        send_cw = pltpu.make_async_remote_copy(
            src_ref=own_ref, dst_ref=chunk_l_ref,
            send_sem=send_sems.at[0], recv_sem=recv_sems.at[0],
            device_id=(right,), device_id_type=pl.DeviceIdType.MESH,
        )
        send_ccw = pltpu.make_async_remote_copy(
            src_ref=own_ref, dst_ref=chunk_r_ref,
            send_sem=send_sems.at[1], recv_sem=recv_sems.at[1],
            device_id=(left,), device_id_type=pl.DeviceIdType.MESH,
        )
        send_cw.start()
        send_ccw.start()

        acc = jnp.dot(ctx, wo_ref[pl.ds(my_pos * D_CTX, D_CTX), :],
                      preferred_element_type=jnp.float32)

        def wo_dot(chunk_half, origin, h):
            return jnp.dot(
                chunk_half,
                wo_ref[pl.ds(origin * D_CTX + h * HALF, HALF), :],
                preferred_element_type=jnp.float32)

        send_cw.wait_recv()
        relay_cw = pltpu.make_async_remote_copy(
            src_ref=chunk_l_ref.at[0], dst_ref=half_cw_ref,
            send_sem=send_sems.at[2], recv_sem=recv_sems.at[2],
            device_id=(right,), device_id_type=pl.DeviceIdType.MESH,
        )
        relay_cw.start()
        acc = acc + wo_dot(chunk_l_ref[0], left, 0) + wo_dot(chunk_l_ref[1], left, 1)

        send_ccw.wait_recv()
        relay_ccw = pltpu.make_async_remote_copy(
            src_ref=chunk_r_ref.at[1], dst_ref=half_ccw_ref,
            send_sem=send_sems.at[3], recv_sem=recv_sems.at[3],
            device_id=(left,), device_id_type=pl.DeviceIdType.MESH,
        )
        relay_ccw.start()
        acc = acc + wo_dot(chunk_r_ref[0], right, 0) + wo_dot(chunk_r_ref[1], right, 1)

        opp = lax.rem(my_pos + 2, N_DEV)
        relay_cw.wait_recv()
        acc = acc + wo_dot(half_cw_ref[...], opp, 0)
        relay_ccw.wait_recv()
        acc = acc + wo_dot(half_ccw_ref[...], opp, 1)

        out_ref[...] = acc

        send_cw.wait_send()
        send_ccw.wait_send()
        relay_cw.wait_send()
        relay_ccw.wait_send()

    out2d = pl.pallas_call(
        body,
        out_shape=jax.ShapeDtypeStruct((B * SQ, D_MODEL), jnp.float32),
        in_specs=[pl.BlockSpec(memory_space=pltpu.VMEM)] * 5,
        out_specs=pl.BlockSpec(memory_space=pltpu.VMEM),
        scratch_shapes=[
            pltpu.VMEM((2, B * SQ, HALF), jnp.bfloat16),
            pltpu.VMEM((2, B * SQ, HALF), jnp.bfloat16),
            pltpu.VMEM((2, B * SQ, HALF), jnp.bfloat16),
            pltpu.VMEM((B * SQ, HALF), jnp.bfloat16),
            pltpu.VMEM((B * SQ, HALF), jnp.bfloat16),
            pltpu.SemaphoreType.DMA((4,)),
            pltpu.SemaphoreType.DMA((4,)),
        ],
        compiler_params=pltpu.CompilerParams(collective_id=0),
    )(x2d, Wq_loc, Kt, Vt, Wo)
    return out2d.reshape(B, SQ, D_MODEL)
